# Initial kernel scaffold; baseline (speedup 1.0000x reference)
import jax
import jax.numpy as jnp
from jax.experimental import pallas as pl


def kernel(inputs, embeddings):
    raise NotImplementedError("write your pallas kernel here")


# speedup vs baseline: 1.5313x; 1.5313x over previous
import functools

import jax
import jax.numpy as jnp
from jax import lax
from jax.experimental import pallas as pl

NUM_CODES = 1296
CODE_DIM = 256
K_PAD = 1408
TILE_N = 512


def _vq_body(x_ref, emb_ref, codes_ref, vecs_ref):
    x = x_ref[...]
    emb = emb_ref[...]

    x3 = lax.dot_general(x, emb, (((1,), (1,)), ((), ())),
                         preferred_element_type=jnp.float32)
    x1 = jnp.sum(x * x, axis=1, keepdims=True)
    x2 = jnp.sum(emb * emb, axis=1)[None, :]

    kiota = lax.broadcasted_iota(jnp.int32, (TILE_N, K_PAD), 1)
    pad_mask = jnp.where(kiota >= NUM_CODES, jnp.float32(1e30), jnp.float32(0.0))
    d = x1 + x2 - 2.0 * x3 + pad_mask

    m = jnp.min(d, axis=1, keepdims=True)
    idx = jnp.min(jnp.where(d == m, kiota, jnp.int32(2**31 - 1)), axis=1)
    codes_ref[0, 0, :] = idx

    onehot = (kiota == idx[:, None]).astype(jnp.float32)
    vecs_ref[...] = lax.dot_general(onehot, emb, (((1,), (0,)), ((), ())),
                                    preferred_element_type=jnp.float32)


@jax.jit
def kernel(inputs, embeddings):
    b, m, n, d = inputs.shape
    total = b * m * n
    xf = inputs.reshape(total, d)
    embp = jnp.pad(embeddings, ((0, K_PAD - NUM_CODES), (0, 0)))

    nb = total // TILE_N
    codes3d, vecs = pl.pallas_call(
        _vq_body,
        grid=(nb,),
        in_specs=[
            pl.BlockSpec((TILE_N, d), lambda i: (i, 0)),
            pl.BlockSpec((K_PAD, d), lambda i: (0, 0)),
        ],
        out_specs=[
            pl.BlockSpec((1, 1, TILE_N), lambda i: (i, 0, 0)),
            pl.BlockSpec((TILE_N, d), lambda i: (i, 0)),
        ],
        out_shape=[
            jax.ShapeDtypeStruct((nb, 1, TILE_N), jnp.int32),
            jax.ShapeDtypeStruct((total, d), jnp.float32),
        ],
    )(xf, embp)

    codes = codes3d.reshape(b, m, n)
    code_vecs = vecs.reshape(b, m, n, d)
    return (codes, code_vecs)
